# Initial kernel scaffold; baseline (speedup 1.0000x reference)
import functools

import jax
import jax.numpy as jnp
from jax import lax
from jax.experimental import pallas as pl
from jax.experimental.pallas import tpu as pltpu
from jax.experimental.pallas import tpu_sc as plsc

NN = 10000
NE = 320000
DIN = 128
H = 64
NCLS = 16
CHUNK = 80

_mesh = plsc.VectorSubcoreMesh(
    core_axis_name="c", subcore_axis_name="s", num_cores=2, num_subcores=16
)


@functools.partial(
    pl.kernel,
    out_type=(
        jax.ShapeDtypeStruct((NN,), jnp.float32),
        jax.ShapeDtypeStruct((NN,), jnp.float32),
    ),
    mesh=_mesh,
    scratch_types=[
        pltpu.VMEM((CHUNK,), jnp.int32),
        pltpu.VMEM((CHUNK,), jnp.float32),
        pltpu.VMEM_SHARED((NN,), jnp.float32),
    ],
)
def _deg_kernel(src_hbm, dst_hbm, zeros1_hbm, dout_hbm, din_hbm,
                idx_v, ones_v, deg_sh):
    cid = lax.axis_index("c")
    sid = lax.axis_index("s")

    @pl.when(sid == 0)
    def _():
        pltpu.sync_copy(zeros1_hbm, deg_sh)

    for i in range(CHUNK // 16):
        ones_v[pl.ds(16 * i, 16)] = jnp.ones((16,), jnp.float32)
    plsc.subcore_barrier()

    ept = NE // 16
    steps = ept // CHUNK

    def body(j, carry):
        base = sid * ept + j * CHUNK

        @pl.when(cid == 0)
        def _():
            pltpu.sync_copy(src_hbm.at[pl.ds(base, CHUNK)], idx_v)

        @pl.when(cid == 1)
        def _():
            pltpu.sync_copy(dst_hbm.at[pl.ds(base, CHUNK)], idx_v)

        pltpu.sync_copy(ones_v, deg_sh.at[idx_v], add=True)
        return carry

    lax.fori_loop(0, steps, body, 0)
    plsc.subcore_barrier()

    @pl.when(jnp.logical_and(sid == 0, cid == 0))
    def _():
        pltpu.sync_copy(deg_sh, dout_hbm)

    @pl.when(jnp.logical_and(sid == 0, cid == 1))
    def _():
        pltpu.sync_copy(deg_sh, din_hbm)


@functools.partial(
    pl.kernel,
    out_type=(
        jax.ShapeDtypeStruct((NN, H), jnp.float32),
        jax.ShapeDtypeStruct((NN, H), jnp.float32),
    ),
    mesh=_mesh,
    scratch_types=[
        pltpu.VMEM((CHUNK,), jnp.int32),
        pltpu.VMEM((CHUNK,), jnp.int32),
        pltpu.VMEM((CHUNK, H), jnp.float32),
        pltpu.SemaphoreType.DMA,
        pltpu.VMEM_SHARED((NN, H), jnp.float32),
    ],
)
def _prop_kernel(t_hbm, src_hbm, dst_hbm, zeros2_hbm, a0_hbm, a1_hbm,
                 sidx_v, didx_v, msg_v, sem, agg_sh):
    cid = lax.axis_index("c")
    sid = lax.axis_index("s")

    @pl.when(sid == 0)
    def _():
        pltpu.sync_copy(zeros2_hbm, agg_sh)
    plsc.subcore_barrier()

    per_sc = NE // 2
    ept = per_sc // 16
    steps = ept // CHUNK

    def body(j, carry):
        base = cid * per_sc + sid * ept + j * CHUNK
        pltpu.sync_copy(src_hbm.at[pl.ds(base, CHUNK)], sidx_v)
        pltpu.sync_copy(dst_hbm.at[pl.ds(base, CHUNK)], didx_v)
        pltpu.async_copy(t_hbm.at[sidx_v], msg_v, sem).wait()
        pltpu.sync_copy(msg_v, agg_sh.at[didx_v], add=True)
        return carry

    lax.fori_loop(0, steps, body, 0)
    plsc.subcore_barrier()

    @pl.when(jnp.logical_and(sid == 0, cid == 0))
    def _():
        pltpu.sync_copy(agg_sh, a0_hbm)

    @pl.when(jnp.logical_and(sid == 0, cid == 1))
    def _():
        pltpu.sync_copy(agg_sh, a1_hbm)


BR = 2000


def _stage0_body(x_ref, w_ref, do_ref, di_ref, t_ref, ns_ref, nd_ref):
    ns = lax.rsqrt(jnp.maximum(do_ref[...], 1.0))
    nd = lax.rsqrt(jnp.maximum(di_ref[...], 1.0))
    t_ref[...] = jnp.dot(x_ref[...], w_ref[...],
                         preferred_element_type=jnp.float32) * ns
    ns_ref[...] = ns
    nd_ref[...] = nd


_stage0 = pl.pallas_call(
    _stage0_body,
    grid=(NN // BR,),
    in_specs=[
        pl.BlockSpec((BR, DIN), lambda i: (i, 0)),
        pl.BlockSpec((DIN, H), lambda i: (0, 0)),
        pl.BlockSpec((BR, 1), lambda i: (i, 0)),
        pl.BlockSpec((BR, 1), lambda i: (i, 0)),
    ],
    out_specs=[
        pl.BlockSpec((BR, H), lambda i: (i, 0)),
        pl.BlockSpec((BR, 1), lambda i: (i, 0)),
        pl.BlockSpec((BR, 1), lambda i: (i, 0)),
    ],
    out_shape=[
        jax.ShapeDtypeStruct((NN, H), jnp.float32),
        jax.ShapeDtypeStruct((NN, 1), jnp.float32),
        jax.ShapeDtypeStruct((NN, 1), jnp.float32),
    ],
)


def _mid_body(a0_ref, a1_ref, nd_ref, ns_ref, b_ref, w_ref, t_ref):
    h = jnp.maximum((a0_ref[...] + a1_ref[...]) * nd_ref[...] + b_ref[...], 0.0)
    t_ref[...] = jnp.dot(h, w_ref[...],
                         preferred_element_type=jnp.float32) * ns_ref[...]


_mid = pl.pallas_call(
    _mid_body,
    grid=(NN // BR,),
    in_specs=[
        pl.BlockSpec((BR, H), lambda i: (i, 0)),
        pl.BlockSpec((BR, H), lambda i: (i, 0)),
        pl.BlockSpec((BR, 1), lambda i: (i, 0)),
        pl.BlockSpec((BR, 1), lambda i: (i, 0)),
        pl.BlockSpec((1, H), lambda i: (0, 0)),
        pl.BlockSpec((H, H), lambda i: (0, 0)),
    ],
    out_specs=pl.BlockSpec((BR, H), lambda i: (i, 0)),
    out_shape=jax.ShapeDtypeStruct((NN, H), jnp.float32),
)


def _fin_body(a0_ref, a1_ref, nd_ref, b_ref, wc_ref, bc_ref, o_ref):
    h = jnp.maximum((a0_ref[...] + a1_ref[...]) * nd_ref[...] + b_ref[...], 0.0)
    o_ref[...] = jnp.dot(h, wc_ref[...],
                         preferred_element_type=jnp.float32) + bc_ref[...]


_fin = pl.pallas_call(
    _fin_body,
    grid=(NN // BR,),
    in_specs=[
        pl.BlockSpec((BR, H), lambda i: (i, 0)),
        pl.BlockSpec((BR, H), lambda i: (i, 0)),
        pl.BlockSpec((BR, 1), lambda i: (i, 0)),
        pl.BlockSpec((1, H), lambda i: (0, 0)),
        pl.BlockSpec((H, NCLS), lambda i: (0, 0)),
        pl.BlockSpec((1, NCLS), lambda i: (0, 0)),
    ],
    out_specs=pl.BlockSpec((BR, NCLS), lambda i: (i, 0)),
    out_shape=jax.ShapeDtypeStruct((NN, NCLS), jnp.float32),
)


def kernel(x, edge_index, W1, b1, W2, b2, W3, b3, Wc, bc):
    src = edge_index[0].astype(jnp.int32)
    dst = edge_index[1].astype(jnp.int32)
    z1 = jnp.zeros((NN,), jnp.float32)
    z2 = jnp.zeros((NN, H), jnp.float32)

    dout, din = _deg_kernel(src, dst, z1)
    t1, ns, nd = _stage0(x, W1, dout.reshape(NN, 1), din.reshape(NN, 1))
    a0, a1 = _prop_kernel(t1, src, dst, z2)
    t2 = _mid(a0, a1, nd, ns, b1.reshape(1, H), W2)
    a0, a1 = _prop_kernel(t2, src, dst, z2)
    t3 = _mid(a0, a1, nd, ns, b2.reshape(1, H), W3)
    a0, a1 = _prop_kernel(t3, src, dst, z2)
    return _fin(a0, a1, nd, b3.reshape(1, H), Wc, bc.reshape(1, NCLS))


# speedup vs baseline: 5.2926x; 5.2926x over previous
import functools

import jax
import jax.numpy as jnp
from jax import lax
from jax.experimental import pallas as pl
from jax.experimental.pallas import tpu as pltpu
from jax.experimental.pallas import tpu_sc as plsc

NN = 10000
NE = 320000
DIN = 128
H = 64
NCLS = 16
CHUNK = 80

_mesh = plsc.VectorSubcoreMesh(
    core_axis_name="c", subcore_axis_name="s", num_cores=2, num_subcores=16
)


@functools.partial(
    pl.kernel,
    out_type=(
        jax.ShapeDtypeStruct((NN,), jnp.float32),
        jax.ShapeDtypeStruct((NN,), jnp.float32),
    ),
    mesh=_mesh,
    scratch_types=[
        pltpu.VMEM((CHUNK,), jnp.int32),
        pltpu.VMEM((CHUNK,), jnp.float32),
        pltpu.VMEM_SHARED((NN,), jnp.float32),
    ],
)
def _deg_kernel(src_hbm, dst_hbm, zeros1_hbm, dout_hbm, din_hbm,
                idx_v, ones_v, deg_sh):
    cid = lax.axis_index("c")
    sid = lax.axis_index("s")

    @pl.when(sid == 0)
    def _():
        pltpu.sync_copy(zeros1_hbm, deg_sh)

    for i in range(CHUNK // 16):
        ones_v[pl.ds(16 * i, 16)] = jnp.ones((16,), jnp.float32)
    plsc.subcore_barrier()

    ept = NE // 16
    steps = ept // CHUNK

    def body(j, carry):
        base = sid * ept + j * CHUNK

        @pl.when(cid == 0)
        def _():
            pltpu.sync_copy(src_hbm.at[pl.ds(base, CHUNK)], idx_v)

        @pl.when(cid == 1)
        def _():
            pltpu.sync_copy(dst_hbm.at[pl.ds(base, CHUNK)], idx_v)

        pltpu.sync_copy(ones_v, deg_sh.at[idx_v], add=True)
        return carry

    lax.fori_loop(0, steps, body, 0)
    plsc.subcore_barrier()

    @pl.when(jnp.logical_and(sid == 0, cid == 0))
    def _():
        pltpu.sync_copy(deg_sh, dout_hbm)

    @pl.when(jnp.logical_and(sid == 0, cid == 1))
    def _():
        pltpu.sync_copy(deg_sh, din_hbm)


@functools.partial(
    pl.kernel,
    out_type=(
        jax.ShapeDtypeStruct((NN, H), jnp.float32),
        jax.ShapeDtypeStruct((NN, H), jnp.float32),
    ),
    mesh=_mesh,
    scratch_types=[
        pltpu.VMEM((CHUNK,), jnp.int32),
        pltpu.VMEM((CHUNK,), jnp.int32),
        pltpu.VMEM((CHUNK, H), jnp.float32),
        pltpu.SemaphoreType.DMA,
        pltpu.VMEM_SHARED((NN, H), jnp.float32),
    ],
    compiler_params=pltpu.CompilerParams(use_tc_tiling_on_sc=False),
)
def _prop_kernel(t_hbm, src_hbm, dst_hbm, zeros2_hbm, a0_hbm, a1_hbm,
                 sidx_v, didx_v, msg_v, sem, agg_sh):
    cid = lax.axis_index("c")
    sid = lax.axis_index("s")

    @pl.when(sid == 0)
    def _():
        pltpu.sync_copy(zeros2_hbm, agg_sh)
    plsc.subcore_barrier()

    per_sc = NE // 2
    ept = per_sc // 16
    steps = ept // CHUNK

    def body(j, carry):
        base = cid * per_sc + sid * ept + j * CHUNK
        pltpu.sync_copy(src_hbm.at[pl.ds(base, CHUNK)], sidx_v)
        pltpu.sync_copy(dst_hbm.at[pl.ds(base, CHUNK)], didx_v)
        pltpu.async_copy(t_hbm.at[sidx_v], msg_v, sem).wait()
        pltpu.sync_copy(msg_v, agg_sh.at[didx_v], add=True)
        return carry

    lax.fori_loop(0, steps, body, 0)
    plsc.subcore_barrier()

    @pl.when(jnp.logical_and(sid == 0, cid == 0))
    def _():
        pltpu.sync_copy(agg_sh, a0_hbm)

    @pl.when(jnp.logical_and(sid == 0, cid == 1))
    def _():
        pltpu.sync_copy(agg_sh, a1_hbm)


BR = 2000


def _stage0_body(x_ref, w_ref, do_ref, di_ref, t_ref, ns_ref, nd_ref):
    ns = lax.rsqrt(jnp.maximum(do_ref[...], 1.0))
    nd = lax.rsqrt(jnp.maximum(di_ref[...], 1.0))
    t_ref[...] = jnp.dot(x_ref[...], w_ref[...],
                         preferred_element_type=jnp.float32) * ns
    ns_ref[...] = ns
    nd_ref[...] = nd


_stage0 = pl.pallas_call(
    _stage0_body,
    grid=(NN // BR,),
    in_specs=[
        pl.BlockSpec((BR, DIN), lambda i: (i, 0)),
        pl.BlockSpec((DIN, H), lambda i: (0, 0)),
        pl.BlockSpec((BR, 1), lambda i: (i, 0)),
        pl.BlockSpec((BR, 1), lambda i: (i, 0)),
    ],
    out_specs=[
        pl.BlockSpec((BR, H), lambda i: (i, 0)),
        pl.BlockSpec((BR, 1), lambda i: (i, 0)),
        pl.BlockSpec((BR, 1), lambda i: (i, 0)),
    ],
    out_shape=[
        jax.ShapeDtypeStruct((NN, H), jnp.float32),
        jax.ShapeDtypeStruct((NN, 1), jnp.float32),
        jax.ShapeDtypeStruct((NN, 1), jnp.float32),
    ],
)


def _mid_body(a0_ref, a1_ref, nd_ref, ns_ref, b_ref, w_ref, t_ref):
    h = jnp.maximum((a0_ref[...] + a1_ref[...]) * nd_ref[...] + b_ref[...], 0.0)
    t_ref[...] = jnp.dot(h, w_ref[...],
                         preferred_element_type=jnp.float32) * ns_ref[...]


_mid = pl.pallas_call(
    _mid_body,
    grid=(NN // BR,),
    in_specs=[
        pl.BlockSpec((BR, H), lambda i: (i, 0)),
        pl.BlockSpec((BR, H), lambda i: (i, 0)),
        pl.BlockSpec((BR, 1), lambda i: (i, 0)),
        pl.BlockSpec((BR, 1), lambda i: (i, 0)),
        pl.BlockSpec((1, H), lambda i: (0, 0)),
        pl.BlockSpec((H, H), lambda i: (0, 0)),
    ],
    out_specs=pl.BlockSpec((BR, H), lambda i: (i, 0)),
    out_shape=jax.ShapeDtypeStruct((NN, H), jnp.float32),
)


def _fin_body(a0_ref, a1_ref, nd_ref, b_ref, wc_ref, bc_ref, o_ref):
    h = jnp.maximum((a0_ref[...] + a1_ref[...]) * nd_ref[...] + b_ref[...], 0.0)
    o_ref[...] = jnp.dot(h, wc_ref[...],
                         preferred_element_type=jnp.float32) + bc_ref[...]


_fin = pl.pallas_call(
    _fin_body,
    grid=(NN // BR,),
    in_specs=[
        pl.BlockSpec((BR, H), lambda i: (i, 0)),
        pl.BlockSpec((BR, H), lambda i: (i, 0)),
        pl.BlockSpec((BR, 1), lambda i: (i, 0)),
        pl.BlockSpec((1, H), lambda i: (0, 0)),
        pl.BlockSpec((H, NCLS), lambda i: (0, 0)),
        pl.BlockSpec((1, NCLS), lambda i: (0, 0)),
    ],
    out_specs=pl.BlockSpec((BR, NCLS), lambda i: (i, 0)),
    out_shape=jax.ShapeDtypeStruct((NN, NCLS), jnp.float32),
)


def kernel(x, edge_index, W1, b1, W2, b2, W3, b3, Wc, bc):
    src = edge_index[0].astype(jnp.int32)
    dst = edge_index[1].astype(jnp.int32)
    z1 = jnp.zeros((NN,), jnp.float32)
    z2 = jnp.zeros((NN, H), jnp.float32)

    dout, din = _deg_kernel(src, dst, z1)
    t1, ns, nd = _stage0(x, W1, dout.reshape(NN, 1), din.reshape(NN, 1))
    a0, a1 = _prop_kernel(t1, src, dst, z2)
    t2 = _mid(a0, a1, nd, ns, b1.reshape(1, H), W2)
    a0, a1 = _prop_kernel(t2, src, dst, z2)
    t3 = _mid(a0, a1, nd, ns, b2.reshape(1, H), W3)
    a0, a1 = _prop_kernel(t3, src, dst, z2)
    return _fin(a0, a1, nd, b3.reshape(1, H), Wc, bc.reshape(1, NCLS))


# speedup vs baseline: 17.7188x; 3.3479x over previous
import functools

import jax
import jax.numpy as jnp
from jax import lax
from jax.experimental import pallas as pl
from jax.experimental.pallas import tpu as pltpu
from jax.experimental.pallas import tpu_sc as plsc

NN = 10000
NE = 320000
DIN = 128
H = 64
NCLS = 16
CHUNK = 125
ROWS = NE // CHUNK
NBUF = 4
DGRP = 8

_mesh = plsc.VectorSubcoreMesh(
    core_axis_name="c", subcore_axis_name="s", num_cores=2, num_subcores=16
)


@functools.partial(
    pl.kernel,
    out_type=(
        jax.ShapeDtypeStruct((NN,), jnp.float32),
        jax.ShapeDtypeStruct((NN,), jnp.float32),
    ),
    mesh=_mesh,
    scratch_types=[
        pltpu.VMEM((ROWS // 16, CHUNK), jnp.int32),
        pltpu.VMEM((CHUNK,), jnp.float32),
        pltpu.SemaphoreType.DMA,
        pltpu.VMEM_SHARED((NN,), jnp.float32),
    ],
)
def _deg_kernel(src2_hbm, dst2_hbm, zeros1_hbm, dout_hbm, din_hbm,
                idx_v, ones_v, ssem, deg_sh):
    cid = lax.axis_index("c")
    sid = lax.axis_index("s")
    rpt = ROWS // 16

    @pl.when(sid == 0)
    def _():
        pltpu.sync_copy(zeros1_hbm, deg_sh)

    @pl.when(cid == 0)
    def _():
        pltpu.sync_copy(src2_hbm.at[pl.ds(sid * rpt, rpt), :], idx_v)

    @pl.when(cid == 1)
    def _():
        pltpu.sync_copy(dst2_hbm.at[pl.ds(sid * rpt, rpt), :], idx_v)

    for i in range(CHUNK // 16):
        ones_v[pl.ds(16 * i, 16)] = jnp.ones((16,), jnp.float32)
    ones_v[pl.ds(CHUNK - 16, 16)] = jnp.ones((16,), jnp.float32)
    plsc.subcore_barrier()

    def group(g, carry):
        for b in range(DGRP):
            j = g * DGRP + b
            pltpu.async_copy(ones_v, deg_sh.at[idx_v.at[j]], ssem, add=True)
        for b in range(DGRP):
            j = g * DGRP + b
            pltpu.make_async_copy(ones_v, deg_sh.at[idx_v.at[j]], ssem).wait()
        return carry

    lax.fori_loop(0, rpt // DGRP, group, 0)
    plsc.subcore_barrier()

    @pl.when(jnp.logical_and(sid == 0, cid == 0))
    def _():
        pltpu.sync_copy(deg_sh, dout_hbm)

    @pl.when(jnp.logical_and(sid == 0, cid == 1))
    def _():
        pltpu.sync_copy(deg_sh, din_hbm)


@functools.partial(
    pl.kernel,
    out_type=(
        jax.ShapeDtypeStruct((NN, H), jnp.float32),
        jax.ShapeDtypeStruct((NN, H), jnp.float32),
    ),
    mesh=_mesh,
    scratch_types=[
        pltpu.VMEM((ROWS // 32, CHUNK), jnp.int32),
        pltpu.VMEM((ROWS // 32, CHUNK), jnp.int32),
        pltpu.VMEM((NBUF, CHUNK, H), jnp.float32),
        pltpu.SemaphoreType.DMA,
        pltpu.SemaphoreType.DMA,
        pltpu.VMEM_SHARED((NN, H), jnp.float32),
    ],
    compiler_params=pltpu.CompilerParams(use_tc_tiling_on_sc=False),
)
def _prop_kernel(t_hbm, src2_hbm, dst2_hbm, zeros2_hbm, a0_hbm, a1_hbm,
                 sidx_v, didx_v, msg_v, gsem, ssem, agg_sh):
    cid = lax.axis_index("c")
    sid = lax.axis_index("s")
    rpt = ROWS // 32
    w0 = (cid * 16 + sid) * rpt

    @pl.when(sid == 0)
    def _():
        pltpu.sync_copy(zeros2_hbm, agg_sh)

    pltpu.sync_copy(src2_hbm.at[pl.ds(w0, rpt), :], sidx_v)
    pltpu.sync_copy(dst2_hbm.at[pl.ds(w0, rpt), :], didx_v)
    plsc.subcore_barrier()

    for b in range(NBUF):
        pltpu.async_copy(t_hbm.at[sidx_v.at[b]], msg_v.at[b], gsem)

    def group(g, carry):
        for b in range(NBUF):
            j = g * NBUF + b
            pltpu.make_async_copy(t_hbm.at[sidx_v.at[j]], msg_v.at[b], gsem).wait()
            pltpu.async_copy(msg_v.at[b], agg_sh.at[didx_v.at[j]], ssem, add=True)
            pltpu.make_async_copy(msg_v.at[b], agg_sh.at[didx_v.at[j]], ssem).wait()

            @pl.when(j + NBUF < rpt)
            def _():
                pltpu.async_copy(t_hbm.at[sidx_v.at[j + NBUF]], msg_v.at[b], gsem)
        return carry

    lax.fori_loop(0, rpt // NBUF, group, 0)
    plsc.subcore_barrier()

    @pl.when(jnp.logical_and(sid == 0, cid == 0))
    def _():
        pltpu.sync_copy(agg_sh, a0_hbm)

    @pl.when(jnp.logical_and(sid == 0, cid == 1))
    def _():
        pltpu.sync_copy(agg_sh, a1_hbm)


BR = 2000


def _stage0_body(x_ref, w_ref, do_ref, di_ref, t_ref, ns_ref, nd_ref):
    ns = lax.rsqrt(jnp.maximum(do_ref[...], 1.0))
    nd = lax.rsqrt(jnp.maximum(di_ref[...], 1.0))
    t_ref[...] = jnp.dot(x_ref[...], w_ref[...],
                         preferred_element_type=jnp.float32) * ns
    ns_ref[...] = ns
    nd_ref[...] = nd


_stage0 = pl.pallas_call(
    _stage0_body,
    grid=(NN // BR,),
    in_specs=[
        pl.BlockSpec((BR, DIN), lambda i: (i, 0)),
        pl.BlockSpec((DIN, H), lambda i: (0, 0)),
        pl.BlockSpec((BR, 1), lambda i: (i, 0)),
        pl.BlockSpec((BR, 1), lambda i: (i, 0)),
    ],
    out_specs=[
        pl.BlockSpec((BR, H), lambda i: (i, 0)),
        pl.BlockSpec((BR, 1), lambda i: (i, 0)),
        pl.BlockSpec((BR, 1), lambda i: (i, 0)),
    ],
    out_shape=[
        jax.ShapeDtypeStruct((NN, H), jnp.float32),
        jax.ShapeDtypeStruct((NN, 1), jnp.float32),
        jax.ShapeDtypeStruct((NN, 1), jnp.float32),
    ],
)


def _mid_body(a0_ref, a1_ref, nd_ref, ns_ref, b_ref, w_ref, t_ref):
    h = jnp.maximum((a0_ref[...] + a1_ref[...]) * nd_ref[...] + b_ref[...], 0.0)
    t_ref[...] = jnp.dot(h, w_ref[...],
                         preferred_element_type=jnp.float32) * ns_ref[...]


_mid = pl.pallas_call(
    _mid_body,
    grid=(NN // BR,),
    in_specs=[
        pl.BlockSpec((BR, H), lambda i: (i, 0)),
        pl.BlockSpec((BR, H), lambda i: (i, 0)),
        pl.BlockSpec((BR, 1), lambda i: (i, 0)),
        pl.BlockSpec((BR, 1), lambda i: (i, 0)),
        pl.BlockSpec((1, H), lambda i: (0, 0)),
        pl.BlockSpec((H, H), lambda i: (0, 0)),
    ],
    out_specs=pl.BlockSpec((BR, H), lambda i: (i, 0)),
    out_shape=jax.ShapeDtypeStruct((NN, H), jnp.float32),
)


def _fin_body(a0_ref, a1_ref, nd_ref, b_ref, wc_ref, bc_ref, o_ref):
    h = jnp.maximum((a0_ref[...] + a1_ref[...]) * nd_ref[...] + b_ref[...], 0.0)
    o_ref[...] = jnp.dot(h, wc_ref[...],
                         preferred_element_type=jnp.float32) + bc_ref[...]


_fin = pl.pallas_call(
    _fin_body,
    grid=(NN // BR,),
    in_specs=[
        pl.BlockSpec((BR, H), lambda i: (i, 0)),
        pl.BlockSpec((BR, H), lambda i: (i, 0)),
        pl.BlockSpec((BR, 1), lambda i: (i, 0)),
        pl.BlockSpec((1, H), lambda i: (0, 0)),
        pl.BlockSpec((H, NCLS), lambda i: (0, 0)),
        pl.BlockSpec((1, NCLS), lambda i: (0, 0)),
    ],
    out_specs=pl.BlockSpec((BR, NCLS), lambda i: (i, 0)),
    out_shape=jax.ShapeDtypeStruct((NN, NCLS), jnp.float32),
)


def kernel(x, edge_index, W1, b1, W2, b2, W3, b3, Wc, bc):
    src = edge_index[0].astype(jnp.int32).reshape(ROWS, CHUNK)
    dst = edge_index[1].astype(jnp.int32).reshape(ROWS, CHUNK)
    z1 = jnp.zeros((NN,), jnp.float32)
    z2 = jnp.zeros((NN, H), jnp.float32)

    dout, din = _deg_kernel(src, dst, z1)
    t1, ns, nd = _stage0(x, W1, dout.reshape(NN, 1), din.reshape(NN, 1))
    a0, a1 = _prop_kernel(t1, src, dst, z2)
    t2 = _mid(a0, a1, nd, ns, b1.reshape(1, H), W2)
    a0, a1 = _prop_kernel(t2, src, dst, z2)
    t3 = _mid(a0, a1, nd, ns, b2.reshape(1, H), W3)
    a0, a1 = _prop_kernel(t3, src, dst, z2)
    return _fin(a0, a1, nd, b3.reshape(1, H), Wc, bc.reshape(1, NCLS))
